# Initial kernel scaffold; baseline (speedup 1.0000x reference)
import jax
import jax.numpy as jnp
from jax.experimental import pallas as pl


def kernel(x_piece, edge_index_piece, batch, W1, b1, W2, b2, W3, b3):
    raise NotImplementedError("write your pallas kernel here")


# speedup vs baseline: 11.6281x; 11.6281x over previous
import functools

import jax
import jax.numpy as jnp
from jax import lax
from jax.experimental import pallas as pl
from jax.experimental.pallas import tpu as pltpu
from jax.experimental.pallas import tpu_sc as plsc

_N = 10000
_D = 128
_NPAD = 10240
_NW = 32
_CH = 128
_NCH = 79
_EPAD = _NW * _NCH * _CH
_NPT = _NPAD // 16
_ST = 64


def _sc_degree(dsts):
    mesh = plsc.VectorSubcoreMesh(core_axis_name="c", subcore_axis_name="s")

    @functools.partial(
        pl.kernel,
        out_type=jax.ShapeDtypeStruct((2, _NPAD), jnp.float32),
        mesh=mesh,
        scratch_types=[
            pltpu.VMEM((_NCH, _CH), jnp.int32),
            pltpu.VMEM((_CH,), jnp.float32),
            pltpu.VMEM((_NPT,), jnp.float32),
            pltpu.VMEM_SHARED((_NPAD,), jnp.float32),
        ],
    )
    def k(dsts_hbm, out_hbm, dst_v, ones_v, stage_v, deg_sh):
        c = lax.axis_index("c")
        s = lax.axis_index("s")
        wid = s * 2 + c
        for i in range(_CH // 16):
            ones_v[pl.ds(i * 16, 16)] = jnp.ones((16,), jnp.float32)
        for i in range(_NPT // 16):
            stage_v[pl.ds(i * 16, 16)] = jnp.zeros((16,), jnp.float32)
        pltpu.sync_copy(stage_v, deg_sh.at[pl.ds(s * _NPT, _NPT)])
        plsc.subcore_barrier()
        pltpu.sync_copy(dsts_hbm.at[wid], dst_v)

        def chunk(j, carry):
            pltpu.sync_copy(ones_v, deg_sh.at[dst_v.at[j]], add=True)
            return carry

        lax.fori_loop(0, _NCH, chunk, 0)
        plsc.subcore_barrier()
        pltpu.sync_copy(deg_sh.at[pl.ds(s * _NPT, _NPT)], stage_v)
        pltpu.sync_copy(stage_v, out_hbm.at[c, pl.ds(s * _NPT, _NPT)])

    return k(dsts)


def _sc_edge_agg(hp, srcs, dsts):
    mesh = plsc.VectorSubcoreMesh(core_axis_name="c", subcore_axis_name="s")

    @functools.partial(
        pl.kernel,
        out_type=jax.ShapeDtypeStruct((2, _NPAD, _D), jnp.float32),
        mesh=mesh,
        scratch_types=[
            pltpu.VMEM((_NCH, _CH), jnp.int32),
            pltpu.VMEM((_NCH, _CH), jnp.int32),
            pltpu.VMEM((_CH, _D), jnp.float32),
            pltpu.VMEM((_ST, _D), jnp.float32),
            pltpu.VMEM_SHARED((_NPAD, _D), jnp.float32),
            pltpu.SemaphoreType.DMA,
        ],
    )
    def k(hp_hbm, srcs_hbm, dsts_hbm, out_hbm, src_v, dst_v, rows_v, stage_v,
          acc_sh, sem):
        c = lax.axis_index("c")
        s = lax.axis_index("s")
        wid = s * 2 + c

        def zrow(i, carry):
            for kk in range(_D // 16):
                stage_v[i, pl.ds(kk * 16, 16)] = jnp.zeros((16,), jnp.float32)
            return carry

        lax.fori_loop(0, _ST, zrow, 0)
        for kk in range(_NPT // _ST):
            pltpu.sync_copy(stage_v, acc_sh.at[pl.ds(s * _NPT + kk * _ST, _ST)])
        plsc.subcore_barrier()
        pltpu.sync_copy(srcs_hbm.at[wid], src_v)
        pltpu.sync_copy(dsts_hbm.at[wid], dst_v)

        def chunk(j, carry):
            pltpu.async_copy(hp_hbm.at[src_v.at[j]], rows_v, sem).wait()
            pltpu.sync_copy(rows_v, acc_sh.at[dst_v.at[j]], add=True)
            return carry

        lax.fori_loop(0, _NCH, chunk, 0)
        plsc.subcore_barrier()
        for kk in range(_NPT // _ST):
            sl = pl.ds(s * _NPT + kk * _ST, _ST)
            pltpu.sync_copy(acc_sh.at[sl], stage_v)
            pltpu.sync_copy(stage_v, out_hbm.at[c, sl])

    return k(hp, srcs, dsts)


_BR = 1024


def _gelu(x):
    return 0.5 * x * (1.0 + lax.erf(x * (2.0 ** -0.5)))


def _t1_body(x_ref, deg_ref, w_ref, dis_ref, hp_ref):
    deg = deg_ref[0] + deg_ref[1] + 1.0
    dis = lax.rsqrt(deg)
    dis_ref[...] = dis
    hp_ref[...] = dis * jnp.dot(x_ref[...], w_ref[...],
                                preferred_element_type=jnp.float32)


def _tc_first(xp, degs, w1):
    return pl.pallas_call(
        _t1_body,
        grid=(_NPAD // _BR,),
        in_specs=[
            pl.BlockSpec((_BR, _D), lambda i: (i, 0)),
            pl.BlockSpec((2, _BR, 1), lambda i: (0, i, 0)),
            pl.BlockSpec((_D, _D), lambda i: (0, 0)),
        ],
        out_specs=[
            pl.BlockSpec((_BR, 1), lambda i: (i, 0)),
            pl.BlockSpec((_BR, _D), lambda i: (i, 0)),
        ],
        out_shape=[
            jax.ShapeDtypeStruct((_NPAD, 1), jnp.float32),
            jax.ShapeDtypeStruct((_NPAD, _D), jnp.float32),
        ],
    )(xp, degs, w1)


def _tmid_body(acc_ref, hp_ref, dis_ref, b_ref, w_ref, out_ref):
    ssum = acc_ref[0] + acc_ref[1] + hp_ref[...]
    dis = dis_ref[...]
    pre = dis * ssum + b_ref[...]
    xg = _gelu(pre)
    out_ref[...] = dis * jnp.dot(xg, w_ref[...],
                                 preferred_element_type=jnp.float32)


def _tc_mid(acc, hp, dis, b, w):
    return pl.pallas_call(
        _tmid_body,
        grid=(_NPAD // _BR,),
        in_specs=[
            pl.BlockSpec((2, _BR, _D), lambda i: (0, i, 0)),
            pl.BlockSpec((_BR, _D), lambda i: (i, 0)),
            pl.BlockSpec((_BR, 1), lambda i: (i, 0)),
            pl.BlockSpec((1, _D), lambda i: (0, 0)),
            pl.BlockSpec((_D, _D), lambda i: (0, 0)),
        ],
        out_specs=pl.BlockSpec((_BR, _D), lambda i: (i, 0)),
        out_shape=jax.ShapeDtypeStruct((_NPAD, _D), jnp.float32),
    )(acc, hp, dis, b, w)


def _tfin_body(acc_ref, hp_ref, dis_ref, b_ref, out_ref):
    ssum = acc_ref[0] + acc_ref[1] + hp_ref[...]
    out_ref[...] = dis_ref[...] * ssum + b_ref[...]


def _tc_final(acc, hp, dis, b):
    return pl.pallas_call(
        _tfin_body,
        grid=(_NPAD // _BR,),
        in_specs=[
            pl.BlockSpec((2, _BR, _D), lambda i: (0, i, 0)),
            pl.BlockSpec((_BR, _D), lambda i: (i, 0)),
            pl.BlockSpec((_BR, 1), lambda i: (i, 0)),
            pl.BlockSpec((1, _D), lambda i: (0, 0)),
        ],
        out_specs=pl.BlockSpec((_BR, _D), lambda i: (i, 0)),
        out_shape=jax.ShapeDtypeStruct((_NPAD, _D), jnp.float32),
    )(acc, hp, dis, b)


def kernel(x_piece, edge_index_piece, batch, W1, b1, W2, b2, W3, b3):
    del batch
    src = edge_index_piece[0].astype(jnp.int32)
    dst = edge_index_piece[1].astype(jnp.int32)
    e = src.shape[0]
    pad = _EPAD - e
    srcp = jnp.concatenate([src, jnp.full((pad,), _N, jnp.int32)])
    dstp = jnp.concatenate([dst, jnp.full((pad,), _N, jnp.int32)])
    srcp = srcp.reshape(_NW, _NCH, _CH)
    dstp = dstp.reshape(_NW, _NCH, _CH)
    xp = jnp.concatenate(
        [x_piece, jnp.zeros((_NPAD - _N, _D), jnp.float32)], axis=0)

    degs = _sc_degree(dstp).reshape(2, _NPAD, 1)
    dis, hp1 = _tc_first(xp, degs, W1)
    acc1 = _sc_edge_agg(hp1, srcp, dstp)
    hp2 = _tc_mid(acc1, hp1, dis, b1.reshape(1, _D), W2)
    acc2 = _sc_edge_agg(hp2, srcp, dstp)
    hp3 = _tc_mid(acc2, hp2, dis, b2.reshape(1, _D), W3)
    acc3 = _sc_edge_agg(hp3, srcp, dstp)
    out = _tc_final(acc3, hp3, dis, b3.reshape(1, _D))
    return out[:_N]
